# Initial kernel scaffold; baseline (speedup 1.0000x reference)
import jax
import jax.numpy as jnp
from jax.experimental import pallas as pl


def kernel(text, text_lengths, table, W1, b1, W2, b2):
    raise NotImplementedError("write your pallas kernel here")


# speedup vs baseline: 7.7065x; 7.7065x over previous
import functools

import jax
import jax.numpy as jnp
from jax import lax
from jax.experimental import pallas as pl
from jax.experimental.pallas import tpu as pltpu
from jax.experimental.pallas import tpu_sc as plsc

_NC = 2
_NS = 16
_NW = _NC * _NS
_LANE = 16


def _sc_gather_pool(table, idx_flat, B, L, D):
    b_per_w = B // _NW
    n_idx = b_per_w * L
    c0 = min(128, (L // 2 + 7) // 8 * 8)
    c1 = L - c0
    nd = D // _LANE

    mesh = plsc.VectorSubcoreMesh(core_axis_name="c", subcore_axis_name="s")

    @functools.partial(
        pl.kernel,
        out_type=jax.ShapeDtypeStruct((B, D), jnp.float32),
        mesh=mesh,
        scratch_types=[
            pltpu.VMEM((n_idx,), jnp.int32),
            pltpu.VMEM((c0, D), jnp.float32),
            pltpu.VMEM((c1, D), jnp.float32),
            pltpu.VMEM((b_per_w, D), jnp.float32),
            pltpu.SemaphoreType.DMA,
            pltpu.SemaphoreType.DMA,
        ],
    )
    def pool_kernel(idx_hbm, table_hbm, out_hbm, idx_v, rows0, rows1, out_v,
                    sem0, sem1):
        wid = lax.axis_index("s") * _NC + lax.axis_index("c")
        base = wid * b_per_w
        pltpu.sync_copy(idx_hbm.at[pl.ds(base * L, n_idx)], idx_v)

        def body(b, carry):
            off = pl.multiple_of(b * L, 8)
            cp0 = pltpu.async_copy(
                table_hbm.at[idx_v.at[pl.ds(off, c0)]], rows0, sem0)
            cp1 = pltpu.async_copy(
                table_hbm.at[idx_v.at[pl.ds(off + c0, c1)]], rows1, sem1)
            cp0.wait()
            cp1.wait()

            def acc0(r, acc):
                return tuple(acc[d] + rows0[r, pl.ds(d * _LANE, _LANE)]
                             for d in range(nd))

            def acc1(r, acc):
                return tuple(acc[d] + rows1[r, pl.ds(d * _LANE, _LANE)]
                             for d in range(nd))

            acc = tuple(jnp.zeros((_LANE,), jnp.float32) for _ in range(nd))
            acc = lax.fori_loop(0, c0, acc0, acc)
            acc = lax.fori_loop(0, c1, acc1, acc)
            for d in range(nd):
                out_v[b, pl.ds(d * _LANE, _LANE)] = acc[d]
            return carry

        lax.fori_loop(0, b_per_w, body, 0)
        pltpu.sync_copy(out_v, out_hbm.at[pl.ds(base, b_per_w)])

    return pool_kernel(idx_flat, table)


def _tc_head(pooled_sum, W1, b1, W2p, b2p, inv_l):
    B, D = pooled_sum.shape
    bt = 512

    def head_kernel(p_ref, w1_ref, b1_ref, w2_ref, b2_ref, o_ref):
        p = p_ref[...] * inv_l
        h = lax.dot_general(p, w1_ref[...], (((1,), (1,)), ((), ())),
                            preferred_element_type=jnp.float32,
                            precision=lax.Precision.HIGHEST)
        h = h + b1_ref[...]
        z = lax.dot_general(h, w2_ref[...], (((1,), (1,)), ((), ())),
                            preferred_element_type=jnp.float32,
                            precision=lax.Precision.HIGHEST)
        z = z + b2_ref[...]
        m = jnp.max(z, axis=1, keepdims=True)
        e = jnp.exp(z - m)
        s = jnp.sum(e, axis=1, keepdims=True)
        o_ref[...] = z - (m + jnp.log(s))

    return pl.pallas_call(
        head_kernel,
        grid=(B // bt,),
        in_specs=[
            pl.BlockSpec((bt, D), lambda i: (i, 0)),
            pl.BlockSpec((D, D), lambda i: (0, 0)),
            pl.BlockSpec((1, D), lambda i: (0, 0)),
            pl.BlockSpec((D, D), lambda i: (0, 0)),
            pl.BlockSpec((1, D), lambda i: (0, 0)),
        ],
        out_specs=pl.BlockSpec((bt, D), lambda i: (i, 0)),
        out_shape=jax.ShapeDtypeStruct((B, D), jnp.float32),
    )(pooled_sum, W1, b1, W2p, b2p)


def kernel(text, text_lengths, table, W1, b1, W2, b2):
    B, L = text.shape
    V, D = table.shape
    C = W2.shape[0]

    idx_flat = text.astype(jnp.int32).reshape(B * L)
    pooled_sum = _sc_gather_pool(table, idx_flat, B, L, D)

    W2p = jnp.zeros((D, D), jnp.float32).at[:C].set(W2)
    b2p = jnp.full((1, D), -1e30, jnp.float32).at[0, :C].set(b2)
    b1r = b1.reshape(1, D)

    out_pad = _tc_head(pooled_sum, W1, b1r, W2p, b2p, 1.0 / L)
    return out_pad[:, :C]


# speedup vs baseline: 12.6374x; 1.6398x over previous
import functools

import jax
import jax.numpy as jnp
from jax import lax
from jax.experimental import pallas as pl
from jax.experimental.pallas import tpu as pltpu
from jax.experimental.pallas import tpu_sc as plsc

_NC = 2
_NS = 16
_NW = _NC * _NS
_LANE = 16


def _sc_gather_pool(table, idx_flat, B, L, D):
    b_per_w = B // _NW
    n_idx = b_per_w * L
    c0 = min(128, (L // 2 + 7) // 8 * 8)
    c1 = L - c0
    nd = D // _LANE

    mesh = plsc.VectorSubcoreMesh(core_axis_name="c", subcore_axis_name="s")

    @functools.partial(
        pl.kernel,
        out_type=jax.ShapeDtypeStruct((B, D), jnp.float32),
        mesh=mesh,
        scratch_types=[
            pltpu.VMEM((n_idx,), jnp.int32),
            pltpu.VMEM((c0, D), jnp.float32),
            pltpu.VMEM((c1, D), jnp.float32),
            pltpu.VMEM((c0, D), jnp.float32),
            pltpu.VMEM((c1, D), jnp.float32),
            pltpu.VMEM((b_per_w, D), jnp.float32),
            pltpu.SemaphoreType.DMA,
            pltpu.SemaphoreType.DMA,
        ],
    )
    def pool_kernel(idx_hbm, table_hbm, out_hbm, idx_v, a0, a1, g0, g1, out_v,
                    sem_a, sem_b):
        wid = lax.axis_index("s") * _NC + lax.axis_index("c")
        base = wid * b_per_w
        pltpu.sync_copy(idx_hbm.at[pl.ds(base * L, n_idx)], idx_v)

        def start(r0, r1, sem, b):
            off = pl.multiple_of(b * L, 8)
            pltpu.async_copy(table_hbm.at[idx_v.at[pl.ds(off, c0)]], r0, sem)
            pltpu.async_copy(
                table_hbm.at[idx_v.at[pl.ds(off + c0, c1)]], r1, sem)

        def drain(r0, r1, sem):
            pltpu.make_async_copy(
                table_hbm.at[idx_v.at[pl.ds(0, c0)]], r0, sem).wait()
            pltpu.make_async_copy(
                table_hbm.at[idx_v.at[pl.ds(0, c1)]], r1, sem).wait()

        def accum(r0, r1, b):
            def acc0(r, acc):
                i = 2 * r
                acc = tuple(acc[d] + r0[i, pl.ds(d * _LANE, _LANE)]
                            for d in range(nd))
                return tuple(acc[d] + r0[i + 1, pl.ds(d * _LANE, _LANE)]
                             for d in range(nd))

            def acc1(r, acc):
                i = 2 * r
                acc = tuple(acc[d] + r1[i, pl.ds(d * _LANE, _LANE)]
                            for d in range(nd))
                return tuple(acc[d] + r1[i + 1, pl.ds(d * _LANE, _LANE)]
                             for d in range(nd))

            acc = tuple(jnp.zeros((_LANE,), jnp.float32) for _ in range(nd))
            acc = lax.fori_loop(0, c0 // 2, acc0, acc)
            acc = lax.fori_loop(0, c1 // 2, acc1, acc)
            for d in range(nd):
                out_v[b, pl.ds(d * _LANE, _LANE)] = acc[d]

        start(a0, a1, sem_a, 0)

        def body(i, carry):
            beven = 2 * i
            bodd = beven + 1
            start(g0, g1, sem_b, bodd)
            drain(a0, a1, sem_a)
            accum(a0, a1, beven)

            @pl.when(beven + 2 < b_per_w)
            def _():
                start(a0, a1, sem_a, beven + 2)

            drain(g0, g1, sem_b)
            accum(g0, g1, bodd)
            return carry

        lax.fori_loop(0, b_per_w // 2, body, 0)
        pltpu.sync_copy(out_v, out_hbm.at[pl.ds(base, b_per_w)])

    return pool_kernel(idx_flat, table)


def _tc_head(pooled_sum, W1, b1, W2p, b2p, inv_l):
    B, D = pooled_sum.shape
    bt = 512

    def head_kernel(p_ref, w1_ref, b1_ref, w2_ref, b2_ref, o_ref):
        p = p_ref[...] * inv_l
        h = lax.dot_general(p, w1_ref[...], (((1,), (1,)), ((), ())),
                            preferred_element_type=jnp.float32,
                            precision=lax.Precision.HIGHEST)
        h = h + b1_ref[...]
        z = lax.dot_general(h, w2_ref[...], (((1,), (1,)), ((), ())),
                            preferred_element_type=jnp.float32,
                            precision=lax.Precision.HIGHEST)
        z = z + b2_ref[...]
        m = jnp.max(z, axis=1, keepdims=True)
        e = jnp.exp(z - m)
        s = jnp.sum(e, axis=1, keepdims=True)
        o_ref[...] = z - (m + jnp.log(s))

    return pl.pallas_call(
        head_kernel,
        grid=(B // bt,),
        in_specs=[
            pl.BlockSpec((bt, D), lambda i: (i, 0)),
            pl.BlockSpec((D, D), lambda i: (0, 0)),
            pl.BlockSpec((1, D), lambda i: (0, 0)),
            pl.BlockSpec((D, D), lambda i: (0, 0)),
            pl.BlockSpec((1, D), lambda i: (0, 0)),
        ],
        out_specs=pl.BlockSpec((bt, D), lambda i: (i, 0)),
        out_shape=jax.ShapeDtypeStruct((B, D), jnp.float32),
    )(pooled_sum, W1, b1, W2p, b2p)


def kernel(text, text_lengths, table, W1, b1, W2, b2):
    B, L = text.shape
    V, D = table.shape
    C = W2.shape[0]

    idx_flat = text.astype(jnp.int32).reshape(B * L)
    pooled_sum = _sc_gather_pool(table, idx_flat, B, L, D)

    W2p = jnp.zeros((D, D), jnp.float32).at[:C].set(W2)
    b2p = jnp.full((1, D), -1e30, jnp.float32).at[0, :C].set(b2)
    b1r = b1.reshape(1, D)

    out_pad = _tc_head(pooled_sum, W1, b1r, W2p, b2p, 1.0 / L)
    return out_pad[:, :C]
